# baseline (speedup 1.0000x reference)
import jax
import jax.numpy as jnp
from jax.experimental import pallas as pl
from jax.experimental.pallas import tpu as pltpu

_BN = 512
_NBUF = 4


def _matmul_kernel(x_ref, w_hbm, b_ref, o_ref, wbuf, sems):
    i = pl.program_id(0)
    nsteps = pl.num_programs(0)

    @pl.when(i == 0)
    def _prologue():
        for j in range(_NBUF - 1):
            pltpu.make_async_copy(
                w_hbm.at[pl.ds(j * _BN, _BN), :], wbuf.at[j], sems.at[j]
            ).start()

    nxt = i + _NBUF - 1

    @pl.when(nxt < nsteps)
    def _issue_ahead():
        slot = jax.lax.rem(nxt, _NBUF)
        pltpu.make_async_copy(
            w_hbm.at[pl.ds(nxt * _BN, _BN), :], wbuf.at[slot], sems.at[slot]
        ).start()

    slot = jax.lax.rem(i, _NBUF)
    pltpu.make_async_copy(
        w_hbm.at[pl.ds(i * _BN, _BN), :], wbuf.at[slot], sems.at[slot]
    ).wait()

    wb = wbuf[slot].astype(jnp.bfloat16)
    acc = jax.lax.dot_general(
        x_ref[...], wb,
        dimension_numbers=(((1,), (1,)), ((), ())),
        preferred_element_type=jnp.float32,
    )
    o_ref[...] = acc + b_ref[...]


def kernel(x, W, b):
    M, K = x.shape
    N = W.shape[0]
    xb = x.astype(jnp.bfloat16)
    b2 = b.reshape(1, N)
    out = pl.pallas_call(
        _matmul_kernel,
        grid=(N // _BN,),
        in_specs=[
            pl.BlockSpec((M, K), lambda i: (0, 0)),
            pl.BlockSpec(memory_space=pl.ANY),
            pl.BlockSpec((1, _BN), lambda i: (0, i)),
        ],
        out_specs=pl.BlockSpec((M, _BN), lambda i: (0, i)),
        out_shape=jax.ShapeDtypeStruct((M, N), jnp.float32),
        scratch_shapes=[
            pltpu.VMEM((_NBUF, _BN, K), jnp.float32),
            pltpu.SemaphoreType.DMA((_NBUF,)),
        ],
        compiler_params=pltpu.CompilerParams(
            dimension_semantics=("arbitrary",),
        ),
    )(xb, W, b2)
    return out
